# Initial kernel scaffold; baseline (speedup 1.0000x reference)
import jax
import jax.numpy as jnp
from jax.experimental import pallas as pl


def kernel(token_ids, table):
    raise NotImplementedError("write your pallas kernel here")


# speedup vs baseline: 4.1925x; 4.1925x over previous
import functools

import jax
import jax.numpy as jnp
from jax import lax
from jax.experimental import pallas as pl
from jax.experimental.pallas import tpu as pltpu
from jax.experimental.pallas import tpu_sc as plsc


def _embedding_gather(flat_ids, table, B, D):
    info = plsc.get_sparse_core_info()
    nw = info.num_cores * info.num_subcores
    b_per_w = B // nw
    chunk = 512
    n_chunks = b_per_w // chunk
    assert n_chunks * chunk == b_per_w and n_chunks % 2 == 0

    mesh = plsc.VectorSubcoreMesh(core_axis_name="c", subcore_axis_name="s")

    @functools.partial(
        pl.kernel,
        mesh=mesh,
        out_type=jax.ShapeDtypeStruct((B, D), jnp.float32),
        compiler_params=pltpu.CompilerParams(use_tc_tiling_on_sc=False),
        scratch_types=[
            pltpu.VMEM((chunk,), jnp.int32),
            pltpu.VMEM((chunk,), jnp.int32),
            pltpu.VMEM((chunk, D), jnp.float32),
            pltpu.VMEM((chunk, D), jnp.float32),
            pltpu.SemaphoreType.DMA,
            pltpu.SemaphoreType.DMA,
        ],
    )
    def k(ids_hbm, table_hbm, out_hbm, idx0, idx1, rows0, rows1, sem0, sem1):
        wid = lax.axis_index("s") * info.num_cores + lax.axis_index("c")
        base = wid * b_per_w
        idxs = (idx0, idx1)
        rows = (rows0, rows1)
        sems = (sem0, sem1)

        def start(i, slot):
            off = base + i * chunk
            pltpu.sync_copy(ids_hbm.at[pl.ds(off, chunk)], idxs[slot])
            pltpu.async_copy(table_hbm.at[idxs[slot]], rows[slot], sems[slot])

        def finish(i, slot):
            off = base + i * chunk
            pltpu.make_async_copy(table_hbm.at[idxs[slot]], rows[slot],
                                  sems[slot]).wait()
            pltpu.sync_copy(rows[slot], out_hbm.at[pl.ds(off, chunk)])

        start(0, 0)
        start(1, 1)

        def body(p, carry):
            i = 2 * p
            finish(i, 0)
            start(i + 2, 0)
            finish(i + 1, 1)
            start(i + 3, 1)
            return carry

        lax.fori_loop(0, n_chunks // 2 - 1, body, 0)
        finish(n_chunks - 2, 0)
        finish(n_chunks - 1, 1)

    return k(flat_ids, table)


def kernel(token_ids, table):
    b0, s = token_ids.shape
    v, d = table.shape
    flat_ids = token_ids.reshape(b0 * s).astype(jnp.int32)
    out = _embedding_gather(flat_ids, table, b0 * s, d)
    return out.reshape(b0, s, d)
